# Initial kernel scaffold; baseline (speedup 1.0000x reference)
import jax
import jax.numpy as jnp
from jax.experimental import pallas as pl


def kernel(x, edge_index, W1, b1, W2, b2):
    raise NotImplementedError("write your pallas kernel here")


# speedup vs baseline: 72.6786x; 72.6786x over previous
import functools

import jax
import jax.numpy as jnp
from jax import lax
from jax.experimental import pallas as pl
from jax.experimental.pallas import tpu as pltpu
from jax.experimental.pallas import tpu_sc as plsc

N = 100000
F0, F1, F2 = 16, 32, 2
NP = 100352
E = 3200000
G = 128
NW = 32
PWG = 784
NG = PWG * NW
EP = NG * G
SG = 56
CH = PWG // SG
NB = 4
STEPS = SG // NB
RPT = NP // 16

_mesh = plsc.VectorSubcoreMesh(core_axis_name="c", subcore_axis_name="s")
_sc_params = pltpu.CompilerParams(use_tc_tiling_on_sc=False)


def _zero_rows(buf, nrows):
    z = jnp.zeros((16,), jnp.float32)

    def body(i, _):
        buf[i, :] = z
        return 0

    lax.fori_loop(0, nrows, body, 0)


def _deg_body(dstg, out, acc, dstbuf, zbuf, onesbuf, ssem):
    c = lax.axis_index("c")
    s = lax.axis_index("s")
    wid = s * 2 + c

    z = jnp.zeros((16,), jnp.float32)

    def zb(i, _):
        zbuf[pl.ds(i * 16, 16)] = z
        return 0

    lax.fori_loop(0, RPT // 16, zb, 0)
    o = jnp.ones((16,), jnp.float32)
    for i in range(G // 16):
        onesbuf[pl.ds(i * 16, 16)] = o
    pltpu.sync_copy(zbuf, acc.at[pl.ds(s * RPT, RPT)])
    plsc.subcore_barrier()

    def chunk(t, _):
        gb = wid * PWG + t * SG
        pltpu.sync_copy(dstg.at[pl.ds(gb, SG)], dstbuf)

        def step(si, _):
            @pl.when(si > 0)
            def _drain():
                for _b in range(NB):
                    pltpu.make_async_copy(
                        onesbuf, acc.at[dstbuf.at[0]], ssem).wait()

            for b in range(NB):
                g = si * NB + b
                pltpu.async_copy(
                    onesbuf, acc.at[dstbuf.at[g]], ssem, add=True)
            return 0

        lax.fori_loop(0, STEPS, step, 0)
        for _b in range(NB):
            pltpu.make_async_copy(onesbuf, acc.at[dstbuf.at[0]], ssem).wait()
        return 0

    lax.fori_loop(0, CH, chunk, 0)
    plsc.subcore_barrier()
    pltpu.sync_copy(acc.at[pl.ds(s * RPT, RPT)],
                    out.at[c, pl.ds(s * RPT, RPT)])


def _segsum_body(y, srcg, dstg, out, acc, srcbuf, dstbuf,
                 rb0, rb1, rb2, rb3, gs0, gs1, gs2, gs3):
    c = lax.axis_index("c")
    s = lax.axis_index("s")
    wid = s * 2 + c
    rb = (rb0, rb1, rb2, rb3)
    gs = (gs0, gs1, gs2, gs3)

    _zero_rows(rb0, G)

    def zcp(k, _):
        pltpu.sync_copy(rb0, acc.at[pl.ds(s * RPT + k * G, G)])
        return 0

    lax.fori_loop(0, RPT // G, zcp, 0)
    plsc.subcore_barrier()

    def chunk(t, _):
        gb = wid * PWG + t * SG
        pltpu.sync_copy(srcg.at[pl.ds(gb, SG)], srcbuf)
        pltpu.sync_copy(dstg.at[pl.ds(gb, SG)], dstbuf)
        for b in range(NB):
            pltpu.async_copy(y.at[srcbuf.at[b]], rb[b], gs[b])

        def step(si, _):
            for b in range(NB):
                g = si * NB + b
                pltpu.make_async_copy(y.at[srcbuf.at[g]], rb[b], gs[b]).wait()
                pltpu.sync_copy(rb[b], acc.at[dstbuf.at[g]], add=True)

                @pl.when(si < STEPS - 1)
                def _next():
                    pltpu.async_copy(y.at[srcbuf.at[g + NB]], rb[b], gs[b])
            return 0

        lax.fori_loop(0, STEPS, step, 0)
        return 0

    lax.fori_loop(0, CH, chunk, 0)
    plsc.subcore_barrier()
    pltpu.sync_copy(acc.at[pl.ds(s * RPT, RPT)],
                    out.at[c, pl.ds(s * RPT, RPT)])


_deg_call = pl.kernel(
    _deg_body,
    out_type=jax.ShapeDtypeStruct((2, NP), jnp.float32),
    mesh=_mesh,
    compiler_params=_sc_params,
    scratch_types=[
        pltpu.VMEM_SHARED((NP,), jnp.float32),
        pltpu.VMEM((SG, G), jnp.int32),
        pltpu.VMEM((RPT,), jnp.float32),
        pltpu.VMEM((G,), jnp.float32),
        pltpu.SemaphoreType.DMA,
    ],
)

_segsum_call = pl.kernel(
    _segsum_body,
    out_type=jax.ShapeDtypeStruct((2, NP, F0), jnp.float32),
    mesh=_mesh,
    compiler_params=_sc_params,
    scratch_types=[
        pltpu.VMEM_SHARED((NP, F0), jnp.float32),
        pltpu.VMEM((SG, G), jnp.int32),
        pltpu.VMEM((SG, G), jnp.int32),
        pltpu.VMEM((G, F0), jnp.float32),
        pltpu.VMEM((G, F0), jnp.float32),
        pltpu.VMEM((G, F0), jnp.float32),
        pltpu.VMEM((G, F0), jnp.float32),
        pltpu.SemaphoreType.DMA,
        pltpu.SemaphoreType.DMA,
        pltpu.SemaphoreType.DMA,
        pltpu.SemaphoreType.DMA,
    ],
)


R = 2048
_GRID = NP // R


def _tc_scale_body(degp, xp, d_out, y_out):
    deg = degp[0, :] + degp[1, :] + 1.0
    d = lax.rsqrt(deg)
    d_out[:] = d
    y_out[:, :] = d[:, None] * xp[:, :]


_tc_scale = pl.pallas_call(
    _tc_scale_body,
    grid=(_GRID,),
    in_specs=[
        pl.BlockSpec((2, R), lambda g: (0, g)),
        pl.BlockSpec((R, F0), lambda g: (g, 0)),
    ],
    out_specs=(
        pl.BlockSpec((R,), lambda g: (g,)),
        pl.BlockSpec((R, F0), lambda g: (g, 0)),
    ),
    out_shape=(
        jax.ShapeDtypeStruct((NP,), jnp.float32),
        jax.ShapeDtypeStruct((NP, F0), jnp.float32),
    ),
)


def _tc_dense_body(s1p, xp, d_ref, W1, b1, W2, P_out, q_out):
    d = d_ref[:]
    s1 = s1p[0] + s1p[1]
    agg = d[:, None] * s1 + (d * d)[:, None] * xp[:, :]
    h = jnp.dot(agg, W1[:, :], preferred_element_type=jnp.float32) + b1[:]
    h = jnp.maximum(h, 0.0)
    p = jnp.dot(h, W2[:, :], preferred_element_type=jnp.float32)
    P_out[:, :] = p
    q = d[:, None] * p
    q_out[:, :] = jnp.concatenate(
        [q, jnp.zeros((R, F0 - F2), jnp.float32)], axis=1)


_tc_dense = pl.pallas_call(
    _tc_dense_body,
    grid=(_GRID,),
    in_specs=[
        pl.BlockSpec((2, R, F0), lambda g: (0, g, 0)),
        pl.BlockSpec((R, F0), lambda g: (g, 0)),
        pl.BlockSpec((R,), lambda g: (g,)),
        pl.BlockSpec((F0, F1), lambda g: (0, 0)),
        pl.BlockSpec((F1,), lambda g: (0,)),
        pl.BlockSpec((F1, F2), lambda g: (0, 0)),
    ],
    out_specs=(
        pl.BlockSpec((R, F2), lambda g: (g, 0)),
        pl.BlockSpec((R, F0), lambda g: (g, 0)),
    ),
    out_shape=(
        jax.ShapeDtypeStruct((NP, F2), jnp.float32),
        jax.ShapeDtypeStruct((NP, F0), jnp.float32),
    ),
)


def _tc_final_body(s2p, P_ref, d_ref, b2, out):
    d = d_ref[:]
    s2 = s2p[0, :, 0:F2] + s2p[1, :, 0:F2]
    out[:, :] = d[:, None] * s2 + (d * d)[:, None] * P_ref[:, :] + b2[:]


_tc_final = pl.pallas_call(
    _tc_final_body,
    grid=(_GRID,),
    in_specs=[
        pl.BlockSpec((2, R, F0), lambda g: (0, g, 0)),
        pl.BlockSpec((R, F2), lambda g: (g, 0)),
        pl.BlockSpec((R,), lambda g: (g,)),
        pl.BlockSpec((F2,), lambda g: (0,)),
    ],
    out_specs=pl.BlockSpec((R, F2), lambda g: (g, 0)),
    out_shape=jax.ShapeDtypeStruct((NP, F2), jnp.float32),
)


def kernel(x, edge_index, W1, b1, W2, b2):
    ei = edge_index.astype(jnp.int32)
    pad = EP - E
    fill = jnp.full((pad,), N, jnp.int32)
    srcg = jnp.concatenate([ei[0], fill]).reshape(NG, G)
    dstg = jnp.concatenate([ei[1], fill]).reshape(NG, G)
    xp = jnp.zeros((NP, F0), jnp.float32).at[:N].set(x)

    degp = _deg_call(dstg)
    d, y = _tc_scale(degp, xp)
    s1p = _segsum_call(y, srcg, dstg)
    P, q = _tc_dense(s1p, xp, d, W1, b1, W2)
    s2p = _segsum_call(q, srcg, dstg)
    outp = _tc_final(s2p, P, d, b2)
    return outp[:N]


# speedup vs baseline: 99.1796x; 1.3646x over previous
import functools

import jax
import jax.numpy as jnp
from jax import lax
from jax.experimental import pallas as pl
from jax.experimental.pallas import tpu as pltpu
from jax.experimental.pallas import tpu_sc as plsc

N = 100000
F0, F1, F2 = 16, 32, 2
NP = 100352
NP8 = NP // 8
E = 3200000
G = 128
NW = 32
PWG = 784
NG = PWG * NW
EP = NG * G
SG = 56
CH = PWG // SG
NB = 4
STEPS = SG // NB
RPT = NP // 16

_mesh = plsc.VectorSubcoreMesh(core_axis_name="c", subcore_axis_name="s",
                               num_cores=2, num_subcores=16)
_sc_params = pltpu.CompilerParams(use_tc_tiling_on_sc=False)


def _deg_body(dstg, out, acc, dstbuf, zbuf, onesbuf, ssem):
    c = lax.axis_index("c")
    s = lax.axis_index("s")
    wid = s * 2 + c

    z = jnp.zeros((16,), jnp.float32)

    def zb(i, _):
        zbuf[pl.ds(i * 16, 16)] = z
        return 0

    lax.fori_loop(0, RPT // 16, zb, 0)
    o = jnp.ones((16,), jnp.float32)
    for i in range(G // 16):
        onesbuf[pl.ds(i * 16, 16)] = o
    pltpu.sync_copy(zbuf, acc.at[pl.ds(s * RPT, RPT)])
    plsc.subcore_barrier()

    def chunk(t, _):
        gb = wid * PWG + t * SG
        pltpu.sync_copy(dstg.at[pl.ds(gb, SG)], dstbuf)

        def step(si, _):
            @pl.when(si > 0)
            def _drain():
                for _b in range(NB):
                    pltpu.make_async_copy(
                        onesbuf, acc.at[dstbuf.at[0]], ssem).wait()

            for b in range(NB):
                g = si * NB + b
                pltpu.async_copy(
                    onesbuf, acc.at[dstbuf.at[g]], ssem, add=True)
            return 0

        lax.fori_loop(0, STEPS, step, 0)
        for _b in range(NB):
            pltpu.make_async_copy(onesbuf, acc.at[dstbuf.at[0]], ssem).wait()
        return 0

    lax.fori_loop(0, CH, chunk, 0)
    plsc.subcore_barrier()
    pltpu.sync_copy(acc.at[pl.ds(s * RPT, RPT)],
                    out.at[c, pl.ds(s * RPT, RPT)])


_deg_call = pl.kernel(
    _deg_body,
    out_type=jax.ShapeDtypeStruct((2, NP), jnp.float32),
    mesh=_mesh,
    compiler_params=_sc_params,
    scratch_types=[
        pltpu.VMEM_SHARED((NP,), jnp.float32),
        pltpu.VMEM((SG, G), jnp.int32),
        pltpu.VMEM((RPT,), jnp.float32),
        pltpu.VMEM((G,), jnp.float32),
        pltpu.SemaphoreType.DMA,
    ],
)


def _segsum_main(F, y, srcg, dstg, out, acc, srcbuf, dstbuf, rb, gs):
    c = lax.axis_index("c")
    s = lax.axis_index("s")
    wid = s * 2 + c

    def chunk(t, _):
        gb = wid * PWG + t * SG
        pltpu.sync_copy(srcg.at[pl.ds(gb, SG)], srcbuf)
        pltpu.sync_copy(dstg.at[pl.ds(gb, SG)], dstbuf)
        for b in range(NB):
            pltpu.async_copy(y.at[srcbuf.at[b]], rb[b], gs[b])

        def step(si, _):
            for b in range(NB):
                g = si * NB + b
                pltpu.make_async_copy(y.at[srcbuf.at[g]], rb[b], gs[b]).wait()
                pltpu.sync_copy(rb[b], acc.at[dstbuf.at[g]], add=True)

                @pl.when(si < STEPS - 1)
                def _next():
                    pltpu.async_copy(y.at[srcbuf.at[g + NB]], rb[b], gs[b])
            return 0

        lax.fori_loop(0, STEPS, step, 0)
        return 0

    lax.fori_loop(0, CH, chunk, 0)
    plsc.subcore_barrier()
    pltpu.sync_copy(acc.at[pl.ds(s * RPT, RPT)],
                    out.at[c, pl.ds(s * RPT, RPT)])


def _segsum16_body(y, srcg, dstg, out, acc, srcbuf, dstbuf,
                   rb0, rb1, rb2, rb3, gs0, gs1, gs2, gs3):
    s = lax.axis_index("s")

    z = jnp.zeros((16,), jnp.float32)

    def zr(i, _):
        rb0[i, :] = z
        return 0

    lax.fori_loop(0, G, zr, 0)

    def zcp(k, _):
        pltpu.sync_copy(rb0, acc.at[pl.ds(s * RPT + k * G, G)])
        return 0

    lax.fori_loop(0, RPT // G, zcp, 0)
    plsc.subcore_barrier()
    _segsum_main(F0, y, srcg, dstg, out, acc, srcbuf, dstbuf,
                 (rb0, rb1, rb2, rb3), (gs0, gs1, gs2, gs3))


def _segsum_scratch(F):
    return [
        pltpu.VMEM_SHARED((NP, F), jnp.float32),
        pltpu.VMEM((SG, G), jnp.int32),
        pltpu.VMEM((SG, G), jnp.int32),
        pltpu.VMEM((G, F), jnp.float32),
        pltpu.VMEM((G, F), jnp.float32),
        pltpu.VMEM((G, F), jnp.float32),
        pltpu.VMEM((G, F), jnp.float32),
        pltpu.SemaphoreType.DMA,
        pltpu.SemaphoreType.DMA,
        pltpu.SemaphoreType.DMA,
        pltpu.SemaphoreType.DMA,
    ]


_segsum16_call = pl.kernel(
    _segsum16_body,
    out_type=jax.ShapeDtypeStruct((2, NP, F0), jnp.float32),
    mesh=_mesh,
    compiler_params=_sc_params,
    scratch_types=_segsum_scratch(F0),
)


RB = 1568
_GRID = NP8 // RB


def _tc_d_body(degp, d_out):
    d_out[:, :] = lax.rsqrt(degp[0] + degp[1] + 1.0)


_tc_d = pl.pallas_call(
    _tc_d_body,
    out_shape=jax.ShapeDtypeStruct((NP // 128, 128), jnp.float32),
)


def _tc_y_body(d16, xp, y_out):
    y_out[:, :] = d16[:, :] * xp[:, :]


_tc_y = pl.pallas_call(
    _tc_y_body,
    grid=(_GRID,),
    in_specs=[
        pl.BlockSpec((RB, 128), lambda g: (g, 0)),
        pl.BlockSpec((RB, 128), lambda g: (g, 0)),
    ],
    out_specs=pl.BlockSpec((RB, 128), lambda g: (g, 0)),
    out_shape=jax.ShapeDtypeStruct((NP8, 128), jnp.float32),
)


def _tc_dense_body(s1p, y, d16, W1b, b1b, W2b, P_out, q_out):
    agg = d16[:, :] * (s1p[0] + s1p[1] + y[:, :])
    h = jnp.dot(agg, W1b[:, :], preferred_element_type=jnp.float32) + b1b[:]
    h = jnp.maximum(h, 0.0)
    p = jnp.dot(h, W2b[:, :], preferred_element_type=jnp.float32)
    P_out[:, :] = p
    q_out[:, :] = d16[:, :] * p


_tc_dense = pl.pallas_call(
    _tc_dense_body,
    grid=(_GRID,),
    in_specs=[
        pl.BlockSpec((2, RB, 128), lambda g: (0, g, 0)),
        pl.BlockSpec((RB, 128), lambda g: (g, 0)),
        pl.BlockSpec((RB, 128), lambda g: (g, 0)),
        pl.BlockSpec((128, 8 * F1), lambda g: (0, 0)),
        pl.BlockSpec((8 * F1,), lambda g: (0,)),
        pl.BlockSpec((8 * F1, 128), lambda g: (0, 0)),
    ],
    out_specs=(
        pl.BlockSpec((RB, 128), lambda g: (g, 0)),
        pl.BlockSpec((RB, 128), lambda g: (g, 0)),
    ),
    out_shape=(
        jax.ShapeDtypeStruct((NP8, 128), jnp.float32),
        jax.ShapeDtypeStruct((NP8, 128), jnp.float32),
    ),
)


def _tc_final_body(s2p, P_ref, d16, b2b, out):
    d = d16[:, :]
    out[:, :] = (d * (s2p[0] + s2p[1])
                 + d * d * P_ref[:, :] + b2b[:])


_tc_final = pl.pallas_call(
    _tc_final_body,
    grid=(_GRID,),
    in_specs=[
        pl.BlockSpec((2, RB, 128), lambda g: (0, g, 0)),
        pl.BlockSpec((RB, 128), lambda g: (g, 0)),
        pl.BlockSpec((RB, 128), lambda g: (g, 0)),
        pl.BlockSpec((128,), lambda g: (0,)),
    ],
    out_specs=pl.BlockSpec((RB, 128), lambda g: (g, 0)),
    out_shape=jax.ShapeDtypeStruct((NP8, 128), jnp.float32),
)


def kernel(x, edge_index, W1, b1, W2, b2):
    f32 = jnp.float32
    ei = edge_index.astype(jnp.int32)
    pad = EP - E
    fill = jnp.full((pad,), N, jnp.int32)
    srcg = jnp.concatenate([ei[0], fill]).reshape(NG, G)
    dstg = jnp.concatenate([ei[1], fill]).reshape(NG, G)
    xp128 = jnp.pad(x, ((0, NP - N), (0, 0))).reshape(NP8, 128)

    eye8 = jnp.eye(8, dtype=f32)
    W1b = jnp.kron(eye8, W1)
    b1b = jnp.tile(b1, 8)
    W2b = jnp.kron(eye8, jnp.tile(W2, (1, 8)))
    b2b = jnp.tile(b2, 64)

    degp = _deg_call(dstg)
    d_c = _tc_d(degp.reshape(2, NP // 128, 128))
    d16 = jnp.repeat(d_c.reshape(NP), 16).reshape(NP8, 128)

    y128 = _tc_y(d16, xp128)
    s1p = _segsum16_call(y128.reshape(NP, F0), srcg, dstg)
    Pp, qp = _tc_dense(s1p.reshape(2, NP8, 128), y128, d16,
                       W1b, b1b, W2b)
    s2p = _segsum16_call(qp.reshape(NP, F0), srcg, dstg)
    outp = _tc_final(s2p.reshape(2, NP8, 128), Pp, d16, b2b)
    return outp.reshape(NP, F0)[:N, :F2]


# speedup vs baseline: 110.4402x; 1.1135x over previous
import functools

import jax
import jax.numpy as jnp
from jax import lax
from jax.experimental import pallas as pl
from jax.experimental.pallas import tpu as pltpu
from jax.experimental.pallas import tpu_sc as plsc

N = 100000
F0, F1, F2 = 16, 32, 2
NP = 100352
NP8 = NP // 8
E = 3200000
G = 128
NW = 32
PWG = 784
NG = PWG * NW
EP = NG * G
SG = 56
CH = PWG // SG
NB = 8
STEPS = SG // NB
RPT = NP // 16

_mesh = plsc.VectorSubcoreMesh(core_axis_name="c", subcore_axis_name="s",
                               num_cores=2, num_subcores=16)
_sc_params = pltpu.CompilerParams(use_tc_tiling_on_sc=False)


def _deg_body(dstg, out, acc, dstbuf, zbuf, onesbuf, ssem):
    c = lax.axis_index("c")
    s = lax.axis_index("s")
    wid = s * 2 + c

    z = jnp.zeros((16,), jnp.float32)

    def zb(i, _):
        zbuf[pl.ds(i * 16, 16)] = z
        return 0

    lax.fori_loop(0, RPT // 16, zb, 0)
    o = jnp.ones((16,), jnp.float32)
    for i in range(G // 16):
        onesbuf[pl.ds(i * 16, 16)] = o
    pltpu.sync_copy(zbuf, acc.at[pl.ds(s * RPT, RPT)])
    plsc.subcore_barrier()

    def chunk(t, _):
        gb = wid * PWG + t * SG
        pltpu.sync_copy(dstg.at[pl.ds(gb, SG)], dstbuf)

        def step(si, _):
            @pl.when(si > 0)
            def _drain():
                for _b in range(NB):
                    pltpu.make_async_copy(
                        onesbuf, acc.at[dstbuf.at[0]], ssem).wait()

            for b in range(NB):
                g = si * NB + b
                pltpu.async_copy(
                    onesbuf, acc.at[dstbuf.at[g]], ssem, add=True)
            return 0

        lax.fori_loop(0, STEPS, step, 0)
        for _b in range(NB):
            pltpu.make_async_copy(onesbuf, acc.at[dstbuf.at[0]], ssem).wait()
        return 0

    lax.fori_loop(0, CH, chunk, 0)
    plsc.subcore_barrier()
    pltpu.sync_copy(acc.at[pl.ds(s * RPT, RPT)],
                    out.at[c, pl.ds(s * RPT, RPT)])


_deg_call = pl.kernel(
    _deg_body,
    out_type=jax.ShapeDtypeStruct((2, NP), jnp.float32),
    mesh=_mesh,
    compiler_params=_sc_params,
    scratch_types=[
        pltpu.VMEM_SHARED((NP,), jnp.float32),
        pltpu.VMEM((SG, G), jnp.int32),
        pltpu.VMEM((RPT,), jnp.float32),
        pltpu.VMEM((G,), jnp.float32),
        pltpu.SemaphoreType.DMA,
    ],
)


def _segsum_main(F, y, srcg, dstg, out, acc, srcbuf, dstbuf, rb, gs, ss):
    c = lax.axis_index("c")
    s = lax.axis_index("s")
    wid = s * 2 + c

    def chunk(t, _):
        gb = wid * PWG + t * SG
        pltpu.sync_copy(srcg.at[pl.ds(gb, SG)], srcbuf)
        pltpu.sync_copy(dstg.at[pl.ds(gb, SG)], dstbuf)
        for b in range(NB):
            pltpu.async_copy(y.at[srcbuf.at[b]], rb[b], gs[b])

        def step(si, _):
            for b in range(NB):
                g = si * NB + b
                pltpu.make_async_copy(y.at[srcbuf.at[g]], rb[b], gs[b]).wait()
                pltpu.async_copy(rb[b], acc.at[dstbuf.at[g]], ss[b], add=True)
            for b in range(NB):
                g = si * NB + b
                pltpu.make_async_copy(rb[b], acc.at[dstbuf.at[g]],
                                      ss[b]).wait()

                @pl.when(si < STEPS - 1)
                def _next():
                    pltpu.async_copy(y.at[srcbuf.at[g + NB]], rb[b], gs[b])
            return 0

        lax.fori_loop(0, STEPS, step, 0)
        return 0

    lax.fori_loop(0, CH, chunk, 0)
    plsc.subcore_barrier()
    pltpu.sync_copy(acc.at[pl.ds(s * RPT, RPT)],
                    out.at[c, pl.ds(s * RPT, RPT)])


def _segsum16_body(y, srcg, dstg, out, acc, srcbuf, dstbuf,
                   rb0, rb1, rb2, rb3, rb4, rb5, rb6, rb7,
                   gs0, gs1, gs2, gs3, gs4, gs5, gs6, gs7,
                   ss0, ss1, ss2, ss3, ss4, ss5, ss6, ss7):
    s = lax.axis_index("s")

    z = jnp.zeros((16,), jnp.float32)

    def zr(i, _):
        rb0[i, :] = z
        return 0

    lax.fori_loop(0, G, zr, 0)

    def zcp(k, _):
        pltpu.sync_copy(rb0, acc.at[pl.ds(s * RPT + k * G, G)])
        return 0

    lax.fori_loop(0, RPT // G, zcp, 0)
    plsc.subcore_barrier()
    _segsum_main(F0, y, srcg, dstg, out, acc, srcbuf, dstbuf,
                 (rb0, rb1, rb2, rb3, rb4, rb5, rb6, rb7),
                 (gs0, gs1, gs2, gs3, gs4, gs5, gs6, gs7),
                 (ss0, ss1, ss2, ss3, ss4, ss5, ss6, ss7))


def _segsum_scratch(F):
    return ([
        pltpu.VMEM_SHARED((NP, F), jnp.float32),
        pltpu.VMEM((SG, G), jnp.int32),
        pltpu.VMEM((SG, G), jnp.int32),
    ] + [pltpu.VMEM((G, F), jnp.float32)] * NB
      + [pltpu.SemaphoreType.DMA] * (2 * NB))


_segsum16_call = pl.kernel(
    _segsum16_body,
    out_type=jax.ShapeDtypeStruct((2, NP, F0), jnp.float32),
    mesh=_mesh,
    compiler_params=_sc_params,
    scratch_types=_segsum_scratch(F0),
)


RB = 1568
_GRID = NP8 // RB


def _tc_d_body(degp, d_out):
    d_out[:, :] = lax.rsqrt(degp[0] + degp[1] + 1.0)


_tc_d = pl.pallas_call(
    _tc_d_body,
    out_shape=jax.ShapeDtypeStruct((NP // 128, 128), jnp.float32),
)


def _tc_y_body(d16, xp, y_out):
    y_out[:, :] = d16[:, :] * xp[:, :]


_tc_y = pl.pallas_call(
    _tc_y_body,
    grid=(_GRID,),
    in_specs=[
        pl.BlockSpec((RB, 128), lambda g: (g, 0)),
        pl.BlockSpec((RB, 128), lambda g: (g, 0)),
    ],
    out_specs=pl.BlockSpec((RB, 128), lambda g: (g, 0)),
    out_shape=jax.ShapeDtypeStruct((NP8, 128), jnp.float32),
)


def _tc_dense_body(s1p, y, d16, W1b, b1b, W2b, P_out, q_out):
    agg = d16[:, :] * (s1p[0] + s1p[1] + y[:, :])
    h = jnp.dot(agg, W1b[:, :], preferred_element_type=jnp.float32) + b1b[:]
    h = jnp.maximum(h, 0.0)
    p = jnp.dot(h, W2b[:, :], preferred_element_type=jnp.float32)
    P_out[:, :] = p
    q_out[:, :] = d16[:, :] * p


_tc_dense = pl.pallas_call(
    _tc_dense_body,
    grid=(_GRID,),
    in_specs=[
        pl.BlockSpec((2, RB, 128), lambda g: (0, g, 0)),
        pl.BlockSpec((RB, 128), lambda g: (g, 0)),
        pl.BlockSpec((RB, 128), lambda g: (g, 0)),
        pl.BlockSpec((128, 8 * F1), lambda g: (0, 0)),
        pl.BlockSpec((8 * F1,), lambda g: (0,)),
        pl.BlockSpec((8 * F1, 128), lambda g: (0, 0)),
    ],
    out_specs=(
        pl.BlockSpec((RB, 128), lambda g: (g, 0)),
        pl.BlockSpec((RB, 128), lambda g: (g, 0)),
    ),
    out_shape=(
        jax.ShapeDtypeStruct((NP8, 128), jnp.float32),
        jax.ShapeDtypeStruct((NP8, 128), jnp.float32),
    ),
)


def _tc_final_body(s2p, P_ref, d16, b2b, out):
    d = d16[:, :]
    out[:, :] = (d * (s2p[0] + s2p[1])
                 + d * d * P_ref[:, :] + b2b[:])


_tc_final = pl.pallas_call(
    _tc_final_body,
    grid=(_GRID,),
    in_specs=[
        pl.BlockSpec((2, RB, 128), lambda g: (0, g, 0)),
        pl.BlockSpec((RB, 128), lambda g: (g, 0)),
        pl.BlockSpec((RB, 128), lambda g: (g, 0)),
        pl.BlockSpec((128,), lambda g: (0,)),
    ],
    out_specs=pl.BlockSpec((RB, 128), lambda g: (g, 0)),
    out_shape=jax.ShapeDtypeStruct((NP8, 128), jnp.float32),
)


def kernel(x, edge_index, W1, b1, W2, b2):
    f32 = jnp.float32
    ei = edge_index.astype(jnp.int32)
    pad = EP - E
    fill = jnp.full((pad,), N, jnp.int32)
    srcg = jnp.concatenate([ei[0], fill]).reshape(NG, G)
    dstg = jnp.concatenate([ei[1], fill]).reshape(NG, G)
    xp128 = jnp.pad(x, ((0, NP - N), (0, 0))).reshape(NP8, 128)

    eye8 = jnp.eye(8, dtype=f32)
    W1b = jnp.kron(eye8, W1)
    b1b = jnp.tile(b1, 8)
    W2b = jnp.kron(eye8, jnp.tile(W2, (1, 8)))
    b2b = jnp.tile(b2, 64)

    degp = _deg_call(dstg)
    d_c = _tc_d(degp.reshape(2, NP // 128, 128))
    d16 = jnp.repeat(d_c.reshape(NP), 16).reshape(NP8, 128)

    y128 = _tc_y(d16, xp128)
    s1p = _segsum16_call(y128.reshape(NP, F0), srcg, dstg)
    Pp, qp = _tc_dense(s1p.reshape(2, NP8, 128), y128, d16,
                       W1b, b1b, W2b)
    s2p = _segsum16_call(qp.reshape(NP, F0), srcg, dstg)
    outp = _tc_final(s2p.reshape(2, NP8, 128), Pp, d16, b2b)
    return outp.reshape(NP, F0)[:N, :F2]
